# baseline (speedup 1.0000x reference)
import jax
import jax.numpy as jnp
from jax.experimental import pallas as pl
from jax.experimental.pallas import tpu as pltpu


def _copy_body(src_ref, dst_ref):
    dst_ref[...] = src_ref[...]


def _copy_table(x, block_rows):
    n, d = x.shape
    assert n % block_rows == 0
    return pl.pallas_call(
        _copy_body,
        grid=(n // block_rows,),
        in_specs=[pl.BlockSpec((block_rows, d), lambda i: (i, 0))],
        out_specs=pl.BlockSpec((block_rows, d), lambda i: (i, 0)),
        out_shape=jax.ShapeDtypeStruct(x.shape, x.dtype),
    )(x)


def kernel(W_user, W_item):
    return (_copy_table(W_user, 20000), _copy_table(W_item, 20000))
